# baseline (speedup 1.0000x reference)
import jax
import jax.numpy as jnp
from jax.experimental import pallas as pl
from jax.experimental.pallas import tpu as pltpu

_N = 2048
_HID = 128
_OUT = 64
_NL = 2
_K = 16
_BLK = _N // _K
_NBUF = 6


def _fused_gcn_kernel(x_hbm, adj_hbm, convW_ref, convB_ref, mlpW_ref,
                      mlpB_ref, lnG_ref, lnB_ref, linWT_ref, linB_ref,
                      out_ref, W_s, buf, x_s, xsem, sem):
    f32 = jnp.float32

    def copy(b, slot):
        return pltpu.make_async_copy(
            adj_hbm.at[0, pl.ds(b * _BLK, _BLK), :], buf.at[slot],
            sem.at[slot])

    xcopy = pltpu.make_async_copy(x_hbm.at[0], x_s, xsem)
    xcopy.start()
    for b in range(_NBUF):
        copy(b, b).start()
    xcopy.wait()
    h0T = jax.lax.dot_general(convW_ref[0], x_s[...],
                              (((0,), (1,)), ((), ())),
                              preferred_element_type=f32)
    r_sub = jax.lax.broadcasted_iota(jnp.int32, (_BLK, _BLK), 0)
    c_sub = jax.lax.broadcasted_iota(jnp.int32, (_BLK, _BLK), 1)
    diag_sub = r_sub == c_sub
    ones_blk = jnp.ones((1, _BLK), jnp.bfloat16)
    deg = None
    for b in range(_K):
        slot = b % _NBUF
        copy(b, slot).wait()
        A = buf[slot]
        W_s[pl.ds(b * _BLK, _BLK), :] = jnp.maximum(A, f32(0.0)).astype(
            jnp.bfloat16)
        sub = A[:, b * _BLK:(b + 1) * _BLK]
        wsub = jnp.where(sub > 0, sub,
                         jnp.where(diag_sub, f32(1.0), f32(0.0)))
        W_s[pl.ds(b * _BLK, _BLK), b * _BLK:(b + 1) * _BLK] = wsub.astype(
            jnp.bfloat16)
        part = jnp.dot(ones_blk, W_s[pl.ds(b * _BLK, _BLK), :],
                       preferred_element_type=f32)
        deg = part if deg is None else deg + part
        if b + _NBUF < _K:
            copy(b + _NBUF, slot).start()

    Wb = W_s[...]
    dinv = jax.lax.rsqrt(deg)
    xT = None
    for l in range(_NL):
        if l == 0:
            hT = h0T
        else:
            hT = jax.lax.dot_general(convW_ref[l], xT, (((0,), (0,)), ((), ())),
                                     preferred_element_type=f32)
        hsT = (dinv * hT).astype(jnp.bfloat16)
        aggT = jnp.dot(hsT, Wb, preferred_element_type=f32)
        xT = dinv * aggT + convB_ref[l][:, None]
        xT = jax.lax.dot_general(mlpW_ref[l], xT, (((0,), (0,)), ((), ())),
                                 preferred_element_type=f32)
        xT = xT + mlpB_ref[l][:, None]
        s1 = jnp.sum(xT, axis=0, keepdims=True)
        s2 = jnp.sum(xT * xT, axis=0, keepdims=True)
        mu = s1 * f32(1.0 / _HID)
        var = s2 * f32(1.0 / _HID) - mu * mu
        scale = jax.lax.rsqrt(var + f32(1e-5))
        xT = (xT - mu) * scale * lnG_ref[l][:, None] + lnB_ref[l][:, None]
        xT = jnp.maximum(xT, f32(0.0))
    out_ref[...] = jnp.dot(linWT_ref[...], xT,
                           preferred_element_type=f32) \
        + linB_ref[...][:, None]


def kernel(node_feat, adj, convW, convB, mlpW, mlpB, lnG, lnB, linW, linB):
    vmem = pl.BlockSpec(memory_space=pltpu.MemorySpace.VMEM)
    hbm = pl.BlockSpec(memory_space=pltpu.MemorySpace.HBM)
    out = pl.pallas_call(
        _fused_gcn_kernel,
        in_specs=[
            hbm, hbm,
            vmem, vmem, vmem, vmem, vmem, vmem, vmem, vmem,
        ],
        out_specs=vmem,
        out_shape=jax.ShapeDtypeStruct((_OUT, _N), jnp.float32),
        scratch_shapes=[
            pltpu.VMEM((_N, _N), jnp.bfloat16),
            pltpu.VMEM((_NBUF, _BLK, _N), jnp.float32),
            pltpu.VMEM((_N, _HID), jnp.float32),
            pltpu.SemaphoreType.DMA,
            pltpu.SemaphoreType.DMA((_NBUF,)),
        ],
    )(node_feat, adj, convW, convB, mlpW, mlpB, lnG, lnB, linW.T, linB)
    return out.T[None]


# speedup vs baseline: 1.0092x; 1.0092x over previous
import jax
import jax.numpy as jnp
from jax.experimental import pallas as pl
from jax.experimental.pallas import tpu as pltpu

_N = 2048
_HID = 128
_OUT = 64
_NL = 2
_K = 8
_BLK = _N // _K
_NBUF = 4


def _fused_gcn_kernel(x_hbm, adj_hbm, convW_ref, convB_ref, mlpW_ref,
                      mlpB_ref, lnG_ref, lnB_ref, linWT_ref, linB_ref,
                      out_ref, W_s, buf, x_s, xsem, sem):
    f32 = jnp.float32

    def copy(b, slot):
        return pltpu.make_async_copy(
            adj_hbm.at[0, pl.ds(b * _BLK, _BLK), :], buf.at[slot],
            sem.at[slot])

    xcopy = pltpu.make_async_copy(x_hbm.at[0], x_s, xsem)
    xcopy.start()
    for b in range(_NBUF):
        copy(b, b).start()
    xcopy.wait()
    h0T = jax.lax.dot_general(convW_ref[0], x_s[...],
                              (((0,), (1,)), ((), ())),
                              preferred_element_type=f32)
    r_sub = jax.lax.broadcasted_iota(jnp.int32, (_BLK, _BLK), 0)
    c_sub = jax.lax.broadcasted_iota(jnp.int32, (_BLK, _BLK), 1)
    diag_sub = r_sub == c_sub
    ones_blk = jnp.ones((1, _BLK), jnp.bfloat16)
    deg = None
    for b in range(_K):
        slot = b % _NBUF
        copy(b, slot).wait()
        A = buf[slot]
        W_s[pl.ds(b * _BLK, _BLK), :] = jnp.maximum(A, f32(0.0)).astype(
            jnp.bfloat16)
        sub = A[:, b * _BLK:(b + 1) * _BLK]
        wsub = jnp.where(sub > 0, sub,
                         jnp.where(diag_sub, f32(1.0), f32(0.0)))
        W_s[pl.ds(b * _BLK, _BLK), b * _BLK:(b + 1) * _BLK] = wsub.astype(
            jnp.bfloat16)
        part = jnp.dot(ones_blk, W_s[pl.ds(b * _BLK, _BLK), :],
                       preferred_element_type=f32)
        deg = part if deg is None else deg + part
        if b + _NBUF < _K:
            copy(b + _NBUF, slot).start()

    Wb = W_s[...]
    dinv = jax.lax.rsqrt(deg)
    xT = None
    for l in range(_NL):
        if l == 0:
            hT = h0T
        else:
            hT = jax.lax.dot_general(convW_ref[l], xT, (((0,), (0,)), ((), ())),
                                     preferred_element_type=f32)
        hsT = (dinv * hT).astype(jnp.bfloat16)
        aggT = jnp.dot(hsT, Wb, preferred_element_type=f32)
        xT = dinv * aggT + convB_ref[l][:, None]
        xT = jax.lax.dot_general(mlpW_ref[l], xT, (((0,), (0,)), ((), ())),
                                 preferred_element_type=f32)
        xT = xT + mlpB_ref[l][:, None]
        s1 = jnp.sum(xT, axis=0, keepdims=True)
        s2 = jnp.sum(xT * xT, axis=0, keepdims=True)
        mu = s1 * f32(1.0 / _HID)
        var = s2 * f32(1.0 / _HID) - mu * mu
        scale = jax.lax.rsqrt(var + f32(1e-5))
        xT = (xT - mu) * scale * lnG_ref[l][:, None] + lnB_ref[l][:, None]
        xT = jnp.maximum(xT, f32(0.0))
    out_ref[...] = jnp.dot(linWT_ref[...], xT,
                           preferred_element_type=f32) \
        + linB_ref[...][:, None]


def kernel(node_feat, adj, convW, convB, mlpW, mlpB, lnG, lnB, linW, linB):
    vmem = pl.BlockSpec(memory_space=pltpu.MemorySpace.VMEM)
    hbm = pl.BlockSpec(memory_space=pltpu.MemorySpace.HBM)
    out = pl.pallas_call(
        _fused_gcn_kernel,
        in_specs=[
            hbm, hbm,
            vmem, vmem, vmem, vmem, vmem, vmem, vmem, vmem,
        ],
        out_specs=vmem,
        out_shape=jax.ShapeDtypeStruct((_OUT, _N), jnp.float32),
        scratch_shapes=[
            pltpu.VMEM((_N, _N), jnp.bfloat16),
            pltpu.VMEM((_NBUF, _BLK, _N), jnp.float32),
            pltpu.VMEM((_N, _HID), jnp.float32),
            pltpu.SemaphoreType.DMA,
            pltpu.SemaphoreType.DMA((_NBUF,)),
        ],
    )(node_feat, adj, convW, convB, mlpW, mlpB, lnG, lnB, linW.T, linB)
    return out.T[None]
